# Initial kernel scaffold; baseline (speedup 1.0000x reference)
import jax
import jax.numpy as jnp
from jax.experimental import pallas as pl


def kernel(x, edge_index, edge_weight, W1, b1, W2, b2, W3, b3):
    raise NotImplementedError("write your pallas kernel here")


# speedup vs baseline: 294.6633x; 294.6633x over previous
import jax
import jax.numpy as jnp
from jax import lax
from jax.experimental import pallas as pl
from jax.experimental.pallas import tpu as pltpu
from jax.experimental.pallas import tpu_sc as plsc

N = 10000
E = 320000
D = 128

NC = 2
NS = 16
NW = NC * NS
CHUNK = 128
ACH = 64
EPAD = 327680
EPW = EPAD // NW
NCHUNK = EPW // CHUNK
NACH = EPW // ACH
NGA = ACH // 16
NPAD = 10240
STRIPE1 = NPAD // NS
ROWS_PT = NPAD // NS
NG = CHUNK // 16

_MESH = plsc.VectorSubcoreMesh(core_axis_name="c", subcore_axis_name="s")


def _splat(vec, l):
    idx = jnp.full((16, 1), l, jnp.int32)
    return lax.gather(
        vec, idx,
        dimension_numbers=lax.GatherDimensionNumbers(
            offset_dims=(), collapsed_slice_dims=(0,), start_index_map=(0,)),
        slice_sizes=(1,),
        mode=lax.GatherScatterMode.PROMISE_IN_BOUNDS)


def _deg_body(dst_hbm, ew_hbm, z1_hbm, out_hbm,
              dstv, eww, idxd0, idxd1, dacc, sem0, sem1):
    c = lax.axis_index("c")
    s = lax.axis_index("s")
    wid = s * NC + c
    pltpu.sync_copy(z1_hbm.at[pl.ds(s * STRIPE1, STRIPE1)],
                    dacc.at[pl.ds(s * STRIPE1, STRIPE1)])
    ebase = pl.multiple_of(wid * jnp.int32(EPW), EPW)
    pltpu.sync_copy(dst_hbm.at[pl.ds(ebase, EPW)], dstv)
    pltpu.sync_copy(ew_hbm.at[pl.ds(ebase, EPW)], eww)
    plsc.subcore_barrier()

    def do_chunk(cur, idxd, sem, first):
        cur = pl.multiple_of(cur, CHUNK)
        for q in range(NG):
            idxd[pl.ds(q * 16, 16)] = dstv[pl.ds(cur + q * 16, 16)]

        @pl.when(jnp.logical_not(first))
        def _():
            pltpu.make_async_copy(
                ew_hbm.at[pl.ds(0, CHUNK)], eww.at[pl.ds(0, CHUNK)],
                sem).wait()

        pltpu.async_copy(eww.at[pl.ds(cur, CHUNK)], dacc.at[idxd], sem,
                         add=True)

    def body(i, cur):
        parity = (cur // jnp.int32(CHUNK)) % 2
        first = cur < jnp.int32(2 * CHUNK)

        @pl.when(parity == 0)
        def _():
            do_chunk(cur, idxd0, sem0, first)

        @pl.when(parity == 1)
        def _():
            do_chunk(cur, idxd1, sem1, first)

        return cur + jnp.int32(CHUNK)

    lax.fori_loop(jnp.int32(0), jnp.int32(NCHUNK), body, jnp.int32(0))
    pltpu.make_async_copy(ew_hbm.at[pl.ds(0, CHUNK)],
                          eww.at[pl.ds(0, CHUNK)], sem0).wait()
    pltpu.make_async_copy(ew_hbm.at[pl.ds(0, CHUNK)],
                          eww.at[pl.ds(0, CHUNK)], sem1).wait()
    plsc.subcore_barrier()
    pltpu.sync_copy(dacc.at[pl.ds(s * STRIPE1, STRIPE1)],
                    out_hbm.at[c, pl.ds(s * STRIPE1, STRIPE1)])


_deg = pl.kernel(
    _deg_body,
    out_type=jax.ShapeDtypeStruct((NC, NPAD), jnp.float32),
    mesh=_MESH,
    scratch_types=[
        pltpu.VMEM((EPW,), jnp.int32),
        pltpu.VMEM((EPW,), jnp.float32),
        pltpu.VMEM((CHUNK,), jnp.int32),
        pltpu.VMEM((CHUNK,), jnp.int32),
        pltpu.VMEM_SHARED((NPAD,), jnp.float32),
        pltpu.SemaphoreType.DMA,
        pltpu.SemaphoreType.DMA,
    ],
)


def _agg_body(hp_hbm, z2_hbm, src_hbm, dst_hbm, ew_hbm, out_hbm,
              srcv, dstv, eww, idxd0, idxd1, rows0, rows1,
              acc, gsem0, gsem1, ssem0, ssem1):
    c = lax.axis_index("c")
    s = lax.axis_index("s")
    wid = s * NC + c
    pltpu.sync_copy(z2_hbm.at[pl.ds(s * ROWS_PT, ROWS_PT)],
                    acc.at[pl.ds(s * ROWS_PT, ROWS_PT)])
    ebase = pl.multiple_of(wid * jnp.int32(EPW), EPW)
    pltpu.sync_copy(src_hbm.at[pl.ds(ebase, EPW)], srcv)
    pltpu.sync_copy(dst_hbm.at[pl.ds(ebase, EPW)], dstv)
    pltpu.sync_copy(ew_hbm.at[pl.ds(ebase, EPW)], eww)
    plsc.subcore_barrier()

    pltpu.async_copy(hp_hbm.at[srcv.at[pl.ds(0, ACH)]], rows0, gsem0)

    def scale(rows, cur):
        def grp(g, gcur):
            gcur = pl.multiple_of(gcur, 16)
            ew_vec = eww[pl.ds(gcur, 16)]
            e0 = gcur - cur
            for l in range(16):
                w = _splat(ew_vec, l)
                e = e0 + l
                for j in range(8):
                    sl = pl.ds(j * 16, 16)
                    rows[e, sl] = rows[e, sl] * w
            return gcur + jnp.int32(16)

        lax.fori_loop(jnp.int32(0), jnp.int32(NGA), grp, cur)

    def do_chunk(cur, rows, idxd, gsem, ssem, orows, ogsem, first):
        cur = pl.multiple_of(cur, ACH)
        pltpu.make_async_copy(hp_hbm.at[pl.ds(0, ACH)], rows, gsem).wait()
        nxt = pl.multiple_of(cur + jnp.int32(ACH), ACH)

        @pl.when(nxt < jnp.int32(EPW))
        def _():
            pltpu.async_copy(hp_hbm.at[srcv.at[pl.ds(nxt, ACH)]],
                             orows, ogsem)

        for q in range(NGA):
            idxd[pl.ds(q * 16, 16)] = dstv[pl.ds(cur + q * 16, 16)]
        scale(rows, cur)

        @pl.when(jnp.logical_not(first))
        def _():
            pltpu.make_async_copy(hp_hbm.at[pl.ds(0, ACH)], rows,
                                  ssem).wait()

        pltpu.async_copy(rows, acc.at[idxd], ssem, add=True)

    def body(i, cur):
        parity = (cur // jnp.int32(ACH)) % 2
        first = cur < jnp.int32(2 * ACH)

        @pl.when(parity == 0)
        def _():
            do_chunk(cur, rows0, idxd0, gsem0, ssem0, rows1, gsem1, first)

        @pl.when(parity == 1)
        def _():
            do_chunk(cur, rows1, idxd1, gsem1, ssem1, rows0, gsem0, first)

        return cur + jnp.int32(ACH)

    lax.fori_loop(jnp.int32(0), jnp.int32(NACH), body, jnp.int32(0))
    pltpu.make_async_copy(hp_hbm.at[pl.ds(0, ACH)], rows0, ssem0).wait()
    pltpu.make_async_copy(hp_hbm.at[pl.ds(0, ACH)], rows1, ssem1).wait()
    plsc.subcore_barrier()
    pltpu.sync_copy(acc.at[pl.ds(s * ROWS_PT, ROWS_PT)],
                    out_hbm.at[c, pl.ds(s * ROWS_PT, ROWS_PT)])


_agg = pl.kernel(
    _agg_body,
    out_type=jax.ShapeDtypeStruct((NC, NPAD, D), jnp.float32),
    mesh=_MESH,
    scratch_types=[
        pltpu.VMEM((EPW,), jnp.int32),
        pltpu.VMEM((EPW,), jnp.int32),
        pltpu.VMEM((EPW,), jnp.float32),
        pltpu.VMEM((ACH,), jnp.int32),
        pltpu.VMEM((ACH,), jnp.int32),
        pltpu.VMEM((ACH, D), jnp.float32),
        pltpu.VMEM((ACH, D), jnp.float32),
        pltpu.VMEM_SHARED((NPAD, D), jnp.float32),
        pltpu.SemaphoreType.DMA,
        pltpu.SemaphoreType.DMA,
        pltpu.SemaphoreType.DMA,
        pltpu.SemaphoreType.DMA,
    ],
)


R = 2048
_PREC = jax.lax.Precision.HIGHEST


def _dinv_of(degt_ref):
    return lax.rsqrt(1.0 + degt_ref[:, 0] + degt_ref[:, 1])[:, None]


def _tc1_body(degp_ref, x_ref, w_ref, hp_ref):
    dinv = _dinv_of(degp_ref)
    h = jnp.dot(x_ref[...], w_ref[...],
                preferred_element_type=jnp.float32, precision=_PREC)
    hp_ref[...] = h * dinv


def _tc2_body(p_ref, hp_ref, degp_ref, b_ref, w_ref, o_ref):
    dinv = _dinv_of(degp_ref)
    h1 = jnp.maximum(dinv * (p_ref[0] + p_ref[1] + hp_ref[...]) + b_ref[...],
                     0.0)
    o_ref[...] = dinv * jnp.dot(h1, w_ref[...],
                                preferred_element_type=jnp.float32,
                                precision=_PREC)


def _tc3_body(p_ref, hp_ref, degp_ref, b2_ref, w3_ref, b3_ref, o_ref):
    dinv = _dinv_of(degp_ref)
    h2 = jnp.maximum(dinv * (p_ref[0] + p_ref[1] + hp_ref[...]) + b2_ref[...],
                     0.0)
    o_ref[...] = jnp.tanh(jnp.dot(h2, w3_ref[...],
                                  preferred_element_type=jnp.float32,
                                  precision=_PREC) + b3_ref[...])


_bs_degt = pl.BlockSpec((R, 2), lambda i: (i, jnp.int32(0)))
_bs_rows = pl.BlockSpec((R, D), lambda i: (i, jnp.int32(0)))
_bs_p = pl.BlockSpec((2, R, D), lambda i: (jnp.int32(0), i, jnp.int32(0)))
_bs_w = pl.BlockSpec((D, D), lambda i: (jnp.int32(0), jnp.int32(0)))
_bs_b = pl.BlockSpec((1, D), lambda i: (jnp.int32(0), jnp.int32(0)))

_tc1 = pl.pallas_call(
    _tc1_body,
    grid=(NPAD // R,),
    in_specs=[_bs_degt, _bs_rows, _bs_w],
    out_specs=_bs_rows,
    out_shape=jax.ShapeDtypeStruct((NPAD, D), jnp.float32),
)

_tc2 = pl.pallas_call(
    _tc2_body,
    grid=(NPAD // R,),
    in_specs=[_bs_p, _bs_rows, _bs_degt, _bs_b, _bs_w],
    out_specs=_bs_rows,
    out_shape=jax.ShapeDtypeStruct((NPAD, D), jnp.float32),
)

_tc3 = pl.pallas_call(
    _tc3_body,
    grid=(NPAD // R,),
    in_specs=[_bs_p, _bs_rows, _bs_degt, _bs_b, _bs_w, _bs_b],
    out_specs=_bs_rows,
    out_shape=jax.ShapeDtypeStruct((NPAD, D), jnp.float32),
)


def kernel(x, edge_index, edge_weight, W1, b1, W2, b2, W3, b3):
    x = x.astype(jnp.float32)
    npad_e = EPAD - E
    spread = (jnp.arange(npad_e, dtype=jnp.int32) * 13) % N
    src = jnp.concatenate([edge_index[0].astype(jnp.int32), spread])
    dst = jnp.concatenate([edge_index[1].astype(jnp.int32), spread])
    ew = jnp.concatenate([edge_weight.astype(jnp.float32),
                          jnp.zeros((npad_e,), jnp.float32)])
    z1 = jnp.zeros((NPAD,), jnp.float32)
    z2 = jnp.zeros((NPAD, D), jnp.float32)
    xp = jnp.zeros((NPAD, D), jnp.float32).at[:N].set(x)

    degt = _deg(dst, ew, z1).T
    hp1 = _tc1(degt, xp, W1.astype(jnp.float32))
    p1 = _agg(hp1, z2, src, dst, ew)
    hp2 = _tc2(p1, hp1, degt, b1.reshape(1, D).astype(jnp.float32),
               W2.astype(jnp.float32))
    p2 = _agg(hp2, z2, src, dst, ew)
    out = _tc3(p2, hp2, degt, b2.reshape(1, D).astype(jnp.float32),
               W3.astype(jnp.float32), b3.reshape(1, D).astype(jnp.float32))
    return out[:N].astype(jnp.float64)
